# Initial kernel scaffold; baseline (speedup 1.0000x reference)
import jax
import jax.numpy as jnp
from jax.experimental import pallas as pl


def kernel(word_sequences, embedding_table):
    raise NotImplementedError("write your pallas kernel here")


# speedup vs baseline: 4.5013x; 4.5013x over previous
import functools

import jax
import jax.numpy as jnp
from jax import lax
from jax.experimental import pallas as pl
from jax.experimental.pallas import tpu as pltpu
from jax.experimental.pallas import tpu_sc as plsc

VOCAB = 1000
EMBED_DIM = 128
BATCH = 4096
HIST = 200

_B = BATCH * HIST
_NC = 2
_NS = 16
_NW = _NC * _NS
_PER_W = _B // _NW
_C = 128
_N_CHUNKS = _PER_W // _C


def _gather_kernel(table_hbm, idx_hbm, out_hbm, idx_v, rows_v, sem):
    wid = lax.axis_index("s") * _NC + lax.axis_index("c")
    base = wid * _PER_W

    def step(g, carry):
        off = base + g * _C
        pltpu.sync_copy(idx_hbm.at[pl.ds(off, _C)], idx_v)
        pltpu.async_copy(table_hbm.at[idx_v], rows_v, sem).wait()
        pltpu.sync_copy(rows_v, out_hbm.at[pl.ds(off, _C)])
        return carry

    lax.fori_loop(0, _N_CHUNKS, step, 0)


@jax.jit
def kernel(word_sequences, embedding_table):
    idx_flat = word_sequences.reshape(_B)
    mesh = plsc.VectorSubcoreMesh(core_axis_name="c", subcore_axis_name="s")
    run = pl.kernel(
        _gather_kernel,
        mesh=mesh,
        out_type=jax.ShapeDtypeStruct((_B, EMBED_DIM), jnp.float32),
        scratch_types=[
            pltpu.VMEM((_C,), jnp.int32),
            pltpu.VMEM((_C, EMBED_DIM), jnp.float32),
            pltpu.SemaphoreType.DMA,
        ],
    )
    out = run(embedding_table, idx_flat)
    return out.reshape(BATCH, HIST, EMBED_DIM)


# speedup vs baseline: 5.0054x; 1.1120x over previous
import functools

import jax
import jax.numpy as jnp
from jax import lax
from jax.experimental import pallas as pl
from jax.experimental.pallas import tpu as pltpu
from jax.experimental.pallas import tpu_sc as plsc

VOCAB = 1000
EMBED_DIM = 128
BATCH = 4096
HIST = 200

_B = BATCH * HIST
_NC = 2
_NS = 16
_NW = _NC * _NS
_PER_W = _B // _NW
_C = 128
_NBUF = 4
_N_OUTER = _PER_W // (_C * _NBUF)


def _gather_kernel(table_hbm, idx_hbm, out_hbm, idx_v, rows_v, sem_g, sem_o):
    wid = lax.axis_index("s") * _NC + lax.axis_index("c")
    base = wid * _PER_W

    def outer_body(outer, carry):
        for b in range(_NBUF):
            off = base + (outer * _NBUF + b) * _C

            @pl.when(outer > 0)
            def _reclaim():
                pltpu.make_async_copy(
                    rows_v.at[b], out_hbm.at[pl.ds(base, _C)], sem_o
                ).wait()

            pltpu.sync_copy(idx_hbm.at[pl.ds(off, _C)], idx_v.at[b])
            pltpu.async_copy(table_hbm.at[idx_v.at[b]], rows_v.at[b], sem_g)

        for b in range(_NBUF):
            off = base + (outer * _NBUF + b) * _C
            pltpu.make_async_copy(
                table_hbm.at[idx_v.at[b]], rows_v.at[b], sem_g
            ).wait()
            pltpu.async_copy(rows_v.at[b], out_hbm.at[pl.ds(off, _C)], sem_o)
        return carry

    lax.fori_loop(0, _N_OUTER, outer_body, 0)

    for b in range(_NBUF):
        pltpu.make_async_copy(
            rows_v.at[b], out_hbm.at[pl.ds(base, _C)], sem_o
        ).wait()


@jax.jit
def kernel(word_sequences, embedding_table):
    idx_flat = word_sequences.reshape(_B)
    mesh = plsc.VectorSubcoreMesh(core_axis_name="c", subcore_axis_name="s")
    run = pl.kernel(
        _gather_kernel,
        mesh=mesh,
        out_type=jax.ShapeDtypeStruct((_B, EMBED_DIM), jnp.float32),
        scratch_types=[
            pltpu.VMEM((_NBUF, _C), jnp.int32),
            pltpu.VMEM((_NBUF, _C, EMBED_DIM), jnp.float32),
            pltpu.SemaphoreType.DMA,
            pltpu.SemaphoreType.DMA,
        ],
    )
    out = run(embedding_table, idx_flat)
    return out.reshape(BATCH, HIST, EMBED_DIM)
